# baseline (device time: 123646 ns/iter reference)
import jax
import jax.numpy as jnp
from jax import lax
from jax.experimental import pallas as pl
from jax.experimental.pallas import tpu as pltpu

N_DEV = 4
HB = 8
DH = 128
NG = 4
SCALE = 0.08838834764831843
F32 = jnp.float32
BF16 = jnp.bfloat16


def _body(x_ref, wq_ref, wo_ref, k_ref, v_ref, out_ref,
          comm, kb, vb, ksems, vsems, send_sems, recv_sems):
    my = lax.axis_index("i")
    left = lax.rem(my + N_DEV - 1, N_DEV)
    right = lax.rem(my + 1, N_DEV)

    def fetch_kv(slot, j):
        ck = pltpu.make_async_copy(
            k_ref.at[:, :, pl.ds(j * HB, HB), :], kb.at[slot],
            ksems.at[slot])
        cv = pltpu.make_async_copy(
            v_ref.at[:, :, pl.ds(j * HB, HB), :], vb.at[slot],
            vsems.at[slot])
        ck.start()
        cv.start()
        return ck, cv

    kv_copies = [
        fetch_kv(0, my),
        fetch_kv(1, lax.rem(my + N_DEV - 1, N_DEV)),
        fetch_kv(2, lax.rem(my + 1, N_DEV)),
        fetch_kv(3, lax.rem(my + 2, N_DEV)),
    ]

    barrier = pltpu.get_barrier_semaphore()
    for nbr in (left, right):
        pl.semaphore_signal(barrier, inc=1, device_id=(nbr,),
                            device_id_type=pl.DeviceIdType.MESH)
    pl.semaphore_wait(barrier, 2)

    def attn_block(slot):
        ck, cv = kv_copies[slot]
        ck.wait()
        cv.wait()
        wq_blk = comm[slot, 0]
        ctxs = []
        for r in range(NG):
            x_r = x_ref[r]
            q_r = jnp.dot(x_r, wq_blk, preferred_element_type=F32)
            q_r = q_r.astype(BF16)
            ctx_parts = []
            for h in range(HB):
                q_rh = q_r[:, h * DH:(h + 1) * DH]
                k_rh = kb[slot, r, :, h, :]
                v_rh = vb[slot, r, :, h, :]
                s = lax.dot_general(
                    q_rh, k_rh, (((1,), (1,)), ((), ())),
                    preferred_element_type=F32) * SCALE
                m = jnp.max(s, axis=1, keepdims=True)
                p = jnp.exp(s - m)
                w = (p / jnp.sum(p, axis=1, keepdims=True)).astype(BF16)
                ctx_parts.append(
                    jnp.dot(w, v_rh, preferred_element_type=F32).astype(BF16))
            ctxs.append(jnp.concatenate(ctx_parts, axis=1))
        return ctxs

    def out_block(ctxs, slot, first):
        wo_blk = comm[slot, 1]
        for r in range(NG):
            contrib = jnp.dot(ctxs[r], wo_blk, preferred_element_type=F32)
            for mm in range(4):
                rows = pl.ds(256 * mm + 64 * r, 64)
                piece = contrib[mm * 64:(mm + 1) * 64, :]
                if first:
                    out_ref[rows, :] = piece
                else:
                    out_ref[rows, :] = out_ref[rows, :] + piece

    comm[0, 0] = wq_ref[...]
    comm[0, 1] = wo_ref[...]

    def rdma(src, dst, sem, nbr):
        return pltpu.make_async_remote_copy(
            src_ref=src, dst_ref=dst,
            send_sem=send_sems.at[sem], recv_sem=recv_sems.at[sem],
            device_id=(nbr,), device_id_type=pl.DeviceIdType.MESH,
        )

    a1_r = rdma(comm.at[0, 0], comm.at[1, 0], 0, right)
    a1_l = rdma(comm.at[0, 0], comm.at[2, 0], 1, left)
    a2_r = rdma(comm.at[0, 1], comm.at[1, 1], 2, right)
    a2_l = rdma(comm.at[0, 1], comm.at[2, 1], 3, left)
    a1_r.start()
    a1_l.start()
    a2_r.start()
    a2_l.start()

    ctx0 = attn_block(0)
    out_block(ctx0, 0, first=True)

    a1_r.wait()
    b_r = rdma(comm.at[1, 0], comm.at[3, 0], 4, right)
    b_r.start()
    ctx1 = attn_block(1)

    a1_l.wait()
    ctx2 = attn_block(2)

    a2_l.wait()
    b_l = rdma(comm.at[2, 1], comm.at[3, 1], 5, left)
    b_l.start()
    a2_r.wait()
    out_block(ctx1, 1, first=False)
    out_block(ctx2, 2, first=False)

    b_r.wait()
    ctx3 = attn_block(3)
    b_l.wait()
    out_block(ctx3, 3, first=False)


def kernel(x, Wq, K_ext, V_ext, Wo):
    xg = (x[0].astype(BF16)
          .reshape(4, 4, 64, 1024).transpose(1, 0, 2, 3)
          .reshape(NG, 256, 1024))
    wq = Wq.astype(BF16)
    wo = Wo.astype(BF16)
    kg = (K_ext[0].astype(BF16)
          .reshape(4, 4, 64, 32, 128).transpose(1, 0, 2, 3, 4)
          .reshape(NG, 256, 32, 128))
    vg = (V_ext[0].astype(BF16)
          .reshape(4, 4, 64, 32, 128).transpose(1, 0, 2, 3, 4)
          .reshape(NG, 256, 32, 128))

    out = pl.pallas_call(
        _body,
        out_shape=jax.ShapeDtypeStruct((1024, 1024), F32),
        in_specs=[
            pl.BlockSpec(memory_space=pltpu.VMEM),
            pl.BlockSpec(memory_space=pltpu.VMEM),
            pl.BlockSpec(memory_space=pltpu.VMEM),
            pl.BlockSpec(memory_space=pltpu.MemorySpace.HBM),
            pl.BlockSpec(memory_space=pltpu.MemorySpace.HBM),
        ],
        out_specs=pl.BlockSpec(memory_space=pltpu.VMEM),
        scratch_shapes=[
            pltpu.VMEM((N_DEV, 2, 1024, 1024), BF16),
            pltpu.VMEM((N_DEV, NG, 256, HB, DH), BF16),
            pltpu.VMEM((N_DEV, NG, 256, HB, DH), BF16),
            pltpu.SemaphoreType.DMA((N_DEV,)),
            pltpu.SemaphoreType.DMA((N_DEV,)),
            pltpu.SemaphoreType.DMA((6,)),
            pltpu.SemaphoreType.DMA((6,)),
        ],
        compiler_params=pltpu.CompilerParams(
            collective_id=0, vmem_limit_bytes=100 * 1024 * 1024),
    )(xg, wq, wo, kg, vg)

    return out.reshape(1, 1024, 1024)


# device time: 99924 ns/iter; 1.2374x vs baseline; 1.2374x over previous
import jax
import jax.numpy as jnp
from jax import lax
from jax.experimental import pallas as pl
from jax.experimental.pallas import tpu as pltpu

N_DEV = 4
HB = 8
DH = 128
NG = 4
SCALE = 0.08838834764831843
F32 = jnp.float32
BF16 = jnp.bfloat16


def _body(x_ref, wq_ref, wo_ref, k_ref, v_ref, out_ref,
          comm, kb, vb, ksems, vsems, send_sems, recv_sems):
    my = lax.axis_index("i")
    left = lax.rem(my + N_DEV - 1, N_DEV)
    right = lax.rem(my + 1, N_DEV)

    kv_copies = {}

    def fetch_kv(block, j):
        buf = block % 2
        copies = []
        for h in range(HB):
            jh = j * HB + h
            ck = pltpu.make_async_copy(
                k_ref.at[0, :, jh, :], kb.at[buf, h], ksems.at[buf])
            cv = pltpu.make_async_copy(
                v_ref.at[0, :, jh, :], vb.at[buf, h], vsems.at[buf])
            ck.start()
            cv.start()
            copies.append((ck, cv))
        kv_copies[block] = copies

    blk_j = [my, lax.rem(my + N_DEV - 1, N_DEV),
             lax.rem(my + 1, N_DEV), lax.rem(my + 2, N_DEV)]
    fetch_kv(0, blk_j[0])
    fetch_kv(1, blk_j[1])

    barrier = pltpu.get_barrier_semaphore()
    for nbr in (left, right):
        pl.semaphore_signal(barrier, inc=1, device_id=(nbr,),
                            device_id_type=pl.DeviceIdType.MESH)
    pl.semaphore_wait(barrier, 2)

    def attn_block(block, wq_blk):
        buf = block % 2
        for ck, cv in kv_copies[block]:
            ck.wait()
            cv.wait()
        khg, vhg = [], []
        for h in range(HB):
            kh = kb[buf, h].astype(BF16)
            vh = vb[buf, h].astype(BF16)
            order = [4 * mm + r for r in range(NG) for mm in range(4)]
            khg.append(jnp.concatenate(
                [kh[64 * g:64 * (g + 1), :] for g in order], axis=0))
            vhg.append(jnp.concatenate(
                [vh[64 * g:64 * (g + 1), :] for g in order], axis=0))
        ctxs = []
        for r in range(NG):
            x_r = x_ref[r]
            q_r = jnp.dot(x_r, wq_blk, preferred_element_type=F32)
            q_r = q_r.astype(BF16)
            ctx_parts = []
            for h in range(HB):
                q_rh = q_r[:, h * DH:(h + 1) * DH]
                k_rh = khg[h][256 * r:256 * (r + 1), :]
                v_rh = vhg[h][256 * r:256 * (r + 1), :]
                s = lax.dot_general(
                    q_rh, k_rh, (((1,), (1,)), ((), ())),
                    preferred_element_type=F32) * SCALE
                m = jnp.max(s, axis=1, keepdims=True)
                p = jnp.exp(s - m)
                w = (p / jnp.sum(p, axis=1, keepdims=True)).astype(BF16)
                ctx_parts.append(
                    jnp.dot(w, v_rh, preferred_element_type=F32).astype(BF16))
            ctxs.append(jnp.concatenate(ctx_parts, axis=1))
        return ctxs

    def out_block(ctxs, wo_blk, first):
        for r in range(NG):
            contrib = jnp.dot(ctxs[r], wo_blk, preferred_element_type=F32)
            for mm in range(4):
                rows = pl.ds(256 * mm + 64 * r, 64)
                piece = contrib[mm * 64:(mm + 1) * 64, :]
                if first:
                    out_ref[rows, :] = piece
                else:
                    out_ref[rows, :] = out_ref[rows, :] + piece


    def rdma(src, dst, sem, nbr):
        return pltpu.make_async_remote_copy(
            src_ref=src, dst_ref=dst,
            send_sem=send_sems.at[sem], recv_sem=recv_sems.at[sem],
            device_id=(nbr,), device_id_type=pl.DeviceIdType.MESH,
        )

    a1_r = rdma(wq_ref, comm.at[0, 0], 0, right)
    a1_l = rdma(wq_ref, comm.at[1, 0], 1, left)
    a2_r = rdma(wo_ref, comm.at[0, 1], 2, right)
    a2_l = rdma(wo_ref, comm.at[1, 1], 3, left)
    a1_r.start()
    a1_l.start()
    a2_r.start()
    a2_l.start()

    ctx0 = attn_block(0, wq_ref[...])
    out_block(ctx0, wo_ref[...], first=True)
    fetch_kv(2, blk_j[2])

    a1_r.wait()
    b_r = rdma(comm.at[0, 0], comm.at[2, 0], 4, right)
    b_r.start()
    ctx1 = attn_block(1, comm[0, 0])
    fetch_kv(3, blk_j[3])

    a1_l.wait()
    ctx2 = attn_block(2, comm[1, 0])

    a2_l.wait()
    b_l = rdma(comm.at[1, 1], comm.at[2, 1], 5, left)
    b_l.start()
    a2_r.wait()
    out_block(ctx1, comm[0, 1], first=False)
    out_block(ctx2, comm[1, 1], first=False)

    b_r.wait()
    ctx3 = attn_block(3, comm[2, 0])
    b_l.wait()
    out_block(ctx3, comm[2, 1], first=False)


def kernel(x, Wq, K_ext, V_ext, Wo):
    xg = (x[0].astype(BF16)
          .reshape(4, 4, 64, 1024).transpose(1, 0, 2, 3)
          .reshape(NG, 256, 1024))
    wq = Wq.astype(BF16)
    wo = Wo.astype(BF16)

    out = pl.pallas_call(
        _body,
        out_shape=jax.ShapeDtypeStruct((1024, 1024), F32),
        in_specs=[
            pl.BlockSpec(memory_space=pltpu.VMEM),
            pl.BlockSpec(memory_space=pltpu.VMEM),
            pl.BlockSpec(memory_space=pltpu.VMEM),
            pl.BlockSpec(memory_space=pltpu.MemorySpace.HBM),
            pl.BlockSpec(memory_space=pltpu.MemorySpace.HBM),
        ],
        out_specs=pl.BlockSpec(memory_space=pltpu.VMEM),
        scratch_shapes=[
            pltpu.VMEM((3, 2, 1024, 1024), BF16),
            pltpu.VMEM((2, HB, 1024, DH), F32),
            pltpu.VMEM((2, HB, 1024, DH), F32),
            pltpu.SemaphoreType.DMA((2,)),
            pltpu.SemaphoreType.DMA((2,)),
            pltpu.SemaphoreType.DMA((6,)),
            pltpu.SemaphoreType.DMA((6,)),
        ],
        compiler_params=pltpu.CompilerParams(
            collective_id=0, vmem_limit_bytes=100 * 1024 * 1024),
    )(xg, wq, wo, K_ext, V_ext)

    return out.reshape(1, 1024, 1024)


# device time: 97871 ns/iter; 1.2634x vs baseline; 1.0210x over previous
import jax
import jax.numpy as jnp
from jax import lax
from jax.experimental import pallas as pl
from jax.experimental.pallas import tpu as pltpu

N_DEV = 4
HB = 8
DH = 128
NG = 4
SCALE = 0.08838834764831843
F32 = jnp.float32
BF16 = jnp.bfloat16


def _body(x_ref, wq_ref, wo_ref, k_ref, v_ref, out_ref,
          comm, xg, kb, vb, ksems, vsems, send_sems, recv_sems):
    my = lax.axis_index("i")
    left = lax.rem(my + N_DEV - 1, N_DEV)
    right = lax.rem(my + 1, N_DEV)

    kv_copies = {}

    def fetch_kv(block, j):
        buf = block % 2
        copies = []
        for h in range(HB):
            jh = j * HB + h
            ck = pltpu.make_async_copy(
                k_ref.at[0, :, jh, :], kb.at[buf, h], ksems.at[buf])
            cv = pltpu.make_async_copy(
                v_ref.at[0, :, jh, :], vb.at[buf, h], vsems.at[buf])
            ck.start()
            cv.start()
            copies.append((ck, cv))
        kv_copies[block] = copies

    blk_j = [my, lax.rem(my + N_DEV - 1, N_DEV),
             lax.rem(my + 1, N_DEV), lax.rem(my + 2, N_DEV)]
    fetch_kv(0, blk_j[0])
    fetch_kv(1, blk_j[1])

    barrier = pltpu.get_barrier_semaphore()
    for nbr in (left, right):
        pl.semaphore_signal(barrier, inc=1, device_id=(nbr,),
                            device_id_type=pl.DeviceIdType.MESH)
    for r in range(NG):
        for mm in range(4):
            xg[r, pl.ds(64 * mm, 64), :] = (
                x_ref[0, pl.ds(256 * mm + 64 * r, 64), :].astype(BF16))
    pl.semaphore_wait(barrier, 2)

    def attn_block(block, wq_blk):
        buf = block % 2
        for ck, cv in kv_copies[block]:
            ck.wait()
            cv.wait()
        khg, vhg = [], []
        for h in range(HB):
            kh = kb[buf, h].astype(BF16)
            vh = vb[buf, h].astype(BF16)
            order = [4 * mm + r for r in range(NG) for mm in range(4)]
            khg.append(jnp.concatenate(
                [kh[64 * g:64 * (g + 1), :] for g in order], axis=0))
            vhg.append(jnp.concatenate(
                [vh[64 * g:64 * (g + 1), :] for g in order], axis=0))
        ctxs = []
        for r in range(NG):
            x_r = xg[r]
            q_r = jnp.dot(x_r, wq_blk, preferred_element_type=F32)
            q_r = q_r.astype(BF16)
            ctx_parts = []
            for h in range(HB):
                q_rh = q_r[:, h * DH:(h + 1) * DH]
                k_rh = khg[h][256 * r:256 * (r + 1), :]
                v_rh = vhg[h][256 * r:256 * (r + 1), :]
                s = lax.dot_general(
                    q_rh, k_rh, (((1,), (1,)), ((), ())),
                    preferred_element_type=F32) * SCALE
                m = jnp.max(s, axis=1, keepdims=True)
                p = jnp.exp(s - m)
                w = (p / jnp.sum(p, axis=1, keepdims=True)).astype(BF16)
                ctx_parts.append(
                    jnp.dot(w, v_rh, preferred_element_type=F32).astype(BF16))
            ctxs.append(jnp.concatenate(ctx_parts, axis=1))
        return ctxs

    def out_block(ctxs, wo_blk, first):
        for r in range(NG):
            contrib = jnp.dot(ctxs[r], wo_blk, preferred_element_type=F32)
            for mm in range(4):
                rows = pl.ds(256 * mm + 64 * r, 64)
                piece = contrib[mm * 64:(mm + 1) * 64, :]
                if first:
                    out_ref[rows, :] = piece
                else:
                    out_ref[rows, :] = out_ref[rows, :] + piece


    def rdma(src, dst, sem, nbr):
        return pltpu.make_async_remote_copy(
            src_ref=src, dst_ref=dst,
            send_sem=send_sems.at[sem], recv_sem=recv_sems.at[sem],
            device_id=(nbr,), device_id_type=pl.DeviceIdType.MESH,
        )

    a1_r = rdma(wq_ref, comm.at[0, 0], 0, right)
    a1_l = rdma(wq_ref, comm.at[1, 0], 1, left)
    a2_r = rdma(wo_ref, comm.at[0, 1], 2, right)
    a2_l = rdma(wo_ref, comm.at[1, 1], 3, left)
    a1_r.start()
    a1_l.start()
    a2_r.start()
    a2_l.start()

    ctx0 = attn_block(0, wq_ref[...])
    out_block(ctx0, wo_ref[...], first=True)
    fetch_kv(2, blk_j[2])

    a1_r.wait()
    b_r = rdma(comm.at[0, 0], comm.at[2, 0], 4, right)
    b_r.start()
    ctx1 = attn_block(1, comm[0, 0])
    fetch_kv(3, blk_j[3])

    a1_l.wait()
    ctx2 = attn_block(2, comm[1, 0])

    a2_l.wait()
    b_l = rdma(comm.at[1, 1], comm.at[2, 1], 5, left)
    b_l.start()
    a2_r.wait()
    out_block(ctx1, comm[0, 1], first=False)
    out_block(ctx2, comm[1, 1], first=False)

    b_r.wait()
    ctx3 = attn_block(3, comm[2, 0])
    b_l.wait()
    out_block(ctx3, comm[2, 1], first=False)


def kernel(x, Wq, K_ext, V_ext, Wo):
    wq = Wq.astype(BF16)
    wo = Wo.astype(BF16)

    out = pl.pallas_call(
        _body,
        out_shape=jax.ShapeDtypeStruct((1024, 1024), F32),
        in_specs=[
            pl.BlockSpec(memory_space=pltpu.VMEM),
            pl.BlockSpec(memory_space=pltpu.VMEM),
            pl.BlockSpec(memory_space=pltpu.VMEM),
            pl.BlockSpec(memory_space=pltpu.MemorySpace.HBM),
            pl.BlockSpec(memory_space=pltpu.MemorySpace.HBM),
        ],
        out_specs=pl.BlockSpec(memory_space=pltpu.VMEM),
        scratch_shapes=[
            pltpu.VMEM((3, 2, 1024, 1024), BF16),
            pltpu.VMEM((NG, 256, 1024), BF16),
            pltpu.VMEM((2, HB, 1024, DH), F32),
            pltpu.VMEM((2, HB, 1024, DH), F32),
            pltpu.SemaphoreType.DMA((2,)),
            pltpu.SemaphoreType.DMA((2,)),
            pltpu.SemaphoreType.DMA((6,)),
            pltpu.SemaphoreType.DMA((6,)),
        ],
        compiler_params=pltpu.CompilerParams(
            collective_id=0, vmem_limit_bytes=100 * 1024 * 1024),
    )(x, wq, wo, K_ext, V_ext)

    return out.reshape(1, 1024, 1024)


# device time: 94164 ns/iter; 1.3131x vs baseline; 1.0394x over previous
import jax
import jax.numpy as jnp
from jax import lax
from jax.experimental import pallas as pl
from jax.experimental.pallas import tpu as pltpu

N_DEV = 4
HB = 8
DH = 128
NG = 4
SCALE = 0.08838834764831843
F32 = jnp.float32
BF16 = jnp.bfloat16


def _body(x_ref, wq_ref, wo_ref, k_ref, v_ref, out_ref,
          comm, xg, kb, vb, ksems, vsems, send_sems, recv_sems):
    my = lax.axis_index("i")
    left = lax.rem(my + N_DEV - 1, N_DEV)
    right = lax.rem(my + 1, N_DEV)

    kv_copies = {}

    def fetch_kv(block, j):
        buf = block % 2
        copies = []
        for h in range(HB):
            jh = j * HB + h
            ck = pltpu.make_async_copy(
                k_ref.at[0, :, jh, :], kb.at[buf, h], ksems.at[buf])
            cv = pltpu.make_async_copy(
                v_ref.at[0, :, jh, :], vb.at[buf, h], vsems.at[buf])
            ck.start()
            cv.start()
            copies.append((ck, cv))
        kv_copies[block] = copies

    blk_j = [my, lax.rem(my + N_DEV - 1, N_DEV),
             lax.rem(my + 1, N_DEV), lax.rem(my + 2, N_DEV)]
    fetch_kv(0, blk_j[0])
    fetch_kv(1, blk_j[1])

    barrier = pltpu.get_barrier_semaphore()
    for nbr in (left, right):
        pl.semaphore_signal(barrier, inc=1, device_id=(nbr,),
                            device_id_type=pl.DeviceIdType.MESH)
    for r in range(NG):
        for mm in range(4):
            xg[r, pl.ds(64 * mm, 64), :] = (
                x_ref[0, pl.ds(256 * mm + 64 * r, 64), :].astype(BF16))
    pl.semaphore_wait(barrier, 2)

    def attn_block(block, wq_blk, h_lo=0, n_h=HB):
        buf = block % 2
        for ck, cv in kv_copies[block][h_lo:h_lo + n_h]:
            ck.wait()
            cv.wait()
        khg, vhg = [], []
        for h in range(h_lo, h_lo + n_h):
            kh = kb[buf, h].astype(BF16)
            vh = vb[buf, h].astype(BF16)
            order = [4 * mm + r for r in range(NG) for mm in range(4)]
            khg.append(jnp.concatenate(
                [kh[64 * g:64 * (g + 1), :] for g in order], axis=0))
            vhg.append(jnp.concatenate(
                [vh[64 * g:64 * (g + 1), :] for g in order], axis=0))
        ctxs = []
        for r in range(NG):
            x_r = xg[r]
            q_r = jnp.dot(x_r, wq_blk, preferred_element_type=F32)
            q_r = q_r.astype(BF16)
            ctx_parts = []
            for h in range(n_h):
                q_rh = q_r[:, h * DH:(h + 1) * DH]
                k_rh = khg[h][256 * r:256 * (r + 1), :]
                v_rh = vhg[h][256 * r:256 * (r + 1), :]
                s = lax.dot_general(
                    q_rh, k_rh, (((1,), (1,)), ((), ())),
                    preferred_element_type=F32) * SCALE
                m = jnp.max(s, axis=1, keepdims=True)
                p = jnp.exp(s - m)
                w = (p / jnp.sum(p, axis=1, keepdims=True)).astype(BF16)
                ctx_parts.append(
                    jnp.dot(w, v_rh, preferred_element_type=F32).astype(BF16))
            ctxs.append(jnp.concatenate(ctx_parts, axis=1))
        return ctxs

    def out_block(ctxs, wo_blk, first):
        for r in range(NG):
            contrib = jnp.dot(ctxs[r], wo_blk, preferred_element_type=F32)
            for mm in range(4):
                rows = pl.ds(256 * mm + 64 * r, 64)
                piece = contrib[mm * 64:(mm + 1) * 64, :]
                if first:
                    out_ref[rows, :] = piece
                else:
                    out_ref[rows, :] = out_ref[rows, :] + piece


    def rdma(src, dst, sem, nbr):
        return pltpu.make_async_remote_copy(
            src_ref=src, dst_ref=dst,
            send_sem=send_sems.at[sem], recv_sem=recv_sems.at[sem],
            device_id=(nbr,), device_id_type=pl.DeviceIdType.MESH,
        )

    a1_r = rdma(wq_ref, comm.at[0, 0], 0, right)
    a1_l = rdma(wq_ref, comm.at[1, 0], 1, left)
    a2_r = rdma(wo_ref, comm.at[0, 1], 2, right)
    a2_l = rdma(wo_ref, comm.at[1, 1], 3, left)
    a1_r.start()
    a1_l.start()
    a2_r.start()
    a2_l.start()

    ctx0 = attn_block(0, wq_ref[...])
    out_block(ctx0, wo_ref[...], first=True)
    fetch_kv(2, blk_j[2])

    a1_r.wait()
    b_r1 = rdma(comm.at[0, 0, :, pl.ds(0, 512)],
                comm.at[2, 0, :, pl.ds(0, 512)], 4, right)
    b_r2 = rdma(comm.at[0, 0, :, pl.ds(512, 512)],
                comm.at[2, 0, :, pl.ds(512, 512)], 6, right)
    b_r1.start()
    b_r2.start()
    ctx1 = attn_block(1, comm[0, 0])
    fetch_kv(3, blk_j[3])

    a1_l.wait()
    ctx2 = attn_block(2, comm[1, 0])

    a2_l.wait()
    b_l1 = rdma(comm.at[1, 1, pl.ds(0, 512), :],
                comm.at[2, 1, pl.ds(0, 512), :], 5, left)
    b_l2 = rdma(comm.at[1, 1, pl.ds(512, 512), :],
                comm.at[2, 1, pl.ds(512, 512), :], 7, left)
    b_l1.start()
    b_l2.start()
    a2_r.wait()
    out_block(ctx1, comm[0, 1], first=False)
    out_block(ctx2, comm[1, 1], first=False)

    b_r1.wait()
    ctx3a = attn_block(3, comm[2, 0, :, 0:512], h_lo=0, n_h=4)
    b_l1.wait()
    out_block(ctx3a, comm[2, 1, 0:512, :], first=False)
    b_r2.wait()
    ctx3b = attn_block(3, comm[2, 0, :, 512:1024], h_lo=4, n_h=4)
    b_l2.wait()
    out_block(ctx3b, comm[2, 1, 512:1024, :], first=False)


def kernel(x, Wq, K_ext, V_ext, Wo):
    wq = Wq.astype(BF16)
    wo = Wo.astype(BF16)

    out = pl.pallas_call(
        _body,
        out_shape=jax.ShapeDtypeStruct((1024, 1024), F32),
        in_specs=[
            pl.BlockSpec(memory_space=pltpu.VMEM),
            pl.BlockSpec(memory_space=pltpu.VMEM),
            pl.BlockSpec(memory_space=pltpu.VMEM),
            pl.BlockSpec(memory_space=pltpu.MemorySpace.HBM),
            pl.BlockSpec(memory_space=pltpu.MemorySpace.HBM),
        ],
        out_specs=pl.BlockSpec(memory_space=pltpu.VMEM),
        scratch_shapes=[
            pltpu.VMEM((3, 2, 1024, 1024), BF16),
            pltpu.VMEM((NG, 256, 1024), BF16),
            pltpu.VMEM((2, HB, 1024, DH), F32),
            pltpu.VMEM((2, HB, 1024, DH), F32),
            pltpu.SemaphoreType.DMA((2,)),
            pltpu.SemaphoreType.DMA((2,)),
            pltpu.SemaphoreType.DMA((8,)),
            pltpu.SemaphoreType.DMA((8,)),
        ],
        compiler_params=pltpu.CompilerParams(
            collective_id=0, vmem_limit_bytes=100 * 1024 * 1024),
    )(x, wq, wo, K_ext, V_ext)

    return out.reshape(1, 1024, 1024)
